# baseline (speedup 1.0000x reference)
import jax
import jax.numpy as jnp
from jax import lax
from jax.experimental import pallas as pl

NRAYS = 4096
NS = 64
NI = 128
NF = NS + NI
HID = 256

SPG1 = 4
SPG2 = 4
TR_S = 128
TR_R = 128

F32 = jnp.float32
I32 = jnp.int32


def _sigmoid(x):
    return 1.0 / (1.0 + jnp.exp(-x))


def _mlp_t(pts_t, w1t, b1c, w2t, b2c):
    h = jnp.maximum(
        jnp.dot(w1t, pts_t, preferred_element_type=F32) + b1c, 0.0)
    return jnp.dot(w2t, h, preferred_element_type=F32) + b2c


def _m1_body(ot_ref, dt_ref, bndt_ref, t_ref, w1t_ref, b1c_ref,
             w2t_ref, b2c_ref, rawt_ref):
    ot = ot_ref[...]
    dt = dt_ref[...]
    near = bndt_ref[...][0:1, :]
    far = bndt_ref[...][1:2, :]
    tv = jnp.squeeze(t_ref[...], axis=1)
    cols = []
    for g in range(SPG1):
        tg = tv[g:g + 1, :]
        zc = near * (1.0 - tg) + far * tg
        cols.append(ot + dt * zc)
    pts_t = jnp.concatenate(cols, axis=1)
    rawt_ref[...] = _mlp_t(pts_t, w1t_ref[...], b1c_ref[...],
                           w2t_ref[...], b2c_ref[...])


def _m2_body(ot_ref, dt_ref, zct_ref, w1t_ref, b1c_ref, w2t_ref, b2c_ref,
             rawt_ref):
    ot = ot_ref[...]
    dt = dt_ref[...]
    zc = jnp.squeeze(zct_ref[...], axis=1)
    cols = [ot + dt * zc[g:g + 1, :] for g in range(SPG2)]
    pts_t = jnp.concatenate(cols, axis=1)
    rawt_ref[...] = _mlp_t(pts_t, w1t_ref[...], b1c_ref[...],
                           w2t_ref[...], b2c_ref[...])


def _cumsum_sub(x, n):
    k = 1
    while k < n:
        shifted = jnp.concatenate(
            [jnp.zeros((k,) + x.shape[1:], x.dtype), x[: n - k]], axis=0)
        x = x + shifted
        k *= 2
    return x


def _s_body(d_ref, bnd_ref, t_ref, u_ref, r0_ref, r1_ref, r2_ref, sg_ref,
            rgbc_ref, zf_ref):
    tr = TR_S
    d = d_ref[...]
    near = bnd_ref[...][0:1, :]
    far = bnd_ref[...][1:2, :]
    t = t_ref[...]
    z = near * (1.0 - t) + far * t

    dnorm = jnp.sqrt(jnp.sum(d * d, axis=0, keepdims=True))
    sigma = jnp.maximum(sg_ref[...], 0.0)
    dz = z[1:, :] - z[:-1, :]
    dists = jnp.concatenate([dz, jnp.full((1, tr), 1e10, F32)], axis=0)
    dists = dists * dnorm
    e = jnp.exp(-sigma * dists)
    alpha = 1.0 - e
    lt = jnp.log(e + 1e-10)
    ct_inc = _cumsum_sub(lt, NS)
    ct_exc = jnp.concatenate([jnp.zeros((1, tr), F32), ct_inc[:-1, :]], axis=0)
    trans = jnp.exp(ct_exc)
    weights = alpha * trans
    rgb_rows = [jnp.sum(weights * _sigmoid(rc_ref[...]), axis=0, keepdims=True)
                for rc_ref in (r0_ref, r1_ref, r2_ref)]
    rgbc_ref[...] = jnp.concatenate(rgb_rows, axis=0)

    w = weights + 1e-5
    pdf = w / jnp.sum(w, axis=0, keepdims=True)
    cdf = _cumsum_sub(pdf, NS)
    u = u_ref[...]

    m3 = (cdf[:, None, :] <= u[None, :, :]).astype(F32)
    zpad1 = jnp.zeros((1, tr), F32)
    g1c = jnp.concatenate([pdf[1:, :], zpad1], axis=0)
    b0c = jnp.concatenate([dz, zpad1], axis=0)
    b1c = jnp.concatenate([dz[1:, :], zpad1, zpad1], axis=0)
    g0 = jnp.sum(pdf[:, None, :] * m3, axis=0)
    g1 = cdf[0:1, :] + jnp.sum(g1c[:, None, :] * m3, axis=0)
    bb0 = z[0:1, :] + jnp.sum(b0c[:, None, :] * m3, axis=0)
    bb1 = z[1:2, :] + jnp.sum(b1c[:, None, :] * m3, axis=0)
    denom = g1 - g0
    denom = jnp.where(denom < 1e-5, 1.0, denom)
    tt = (u - g0) / denom
    s = bb0 + tt * (bb1 - bb0)

    big = jnp.full((NS, tr), 3e38, F32)
    c = jnp.concatenate([z, big, s], axis=0)
    row = lax.broadcasted_iota(I32, (256, 1), 0)
    for k in (128, 64, 32, 16, 8, 4, 2, 1):
        keep = (row & k) == 0
        down = jnp.concatenate([c[k:, :], c[:k, :]], axis=0)
        up = jnp.concatenate([c[256 - k:, :], c[:256 - k, :]], axis=0)
        partner = jnp.where(keep, down, up)
        mn = jnp.minimum(c, partner)
        mx = jnp.maximum(c, partner)
        c = jnp.where(keep, mn, mx)
    zf_ref[...] = c[:NF, :]


def _r_body(d_ref, zf_ref, r0_ref, r1_ref, r2_ref, sg_ref,
            rgb_ref, depth_ref, acc_ref, wout_ref):
    tr = TR_R
    d = d_ref[...]
    zf = zf_ref[...]
    dnorm = jnp.sqrt(jnp.sum(d * d, axis=0, keepdims=True))
    sigma = jnp.maximum(sg_ref[...], 0.0)
    dz = zf[1:, :] - zf[:-1, :]
    dists = jnp.concatenate([dz, jnp.full((1, tr), 1e10, F32)], axis=0)
    dists = dists * dnorm
    e = jnp.exp(-sigma * dists)
    alpha = 1.0 - e
    lt = jnp.log(e + 1e-10)
    ct_inc = _cumsum_sub(lt, NF)
    ct_exc = jnp.concatenate([jnp.zeros((1, tr), F32), ct_inc[:-1, :]], axis=0)
    trans = jnp.exp(ct_exc)
    weights = alpha * trans
    rgb_rows = [jnp.sum(weights * _sigmoid(rc_ref[...]), axis=0, keepdims=True)
                for rc_ref in (r0_ref, r1_ref, r2_ref)]
    rgb_ref[...] = jnp.concatenate(rgb_rows, axis=0)
    depth_ref[...] = jnp.sum(weights * zf, axis=0, keepdims=True)
    acc_ref[...] = jnp.sum(weights, axis=0, keepdims=True)
    wout_ref[...] = weights.T


@jax.jit
def kernel(rays_o, rays_d, bounds, W1, b1, W2, b2):
    t_vals = jnp.linspace(0.0, 1.0, NS, dtype=F32).reshape(1, NS)
    t_col = t_vals.reshape(NS, 1)
    u_col = jnp.linspace(0.0, 1.0, NI, dtype=F32)[::-1].reshape(NI, 1)
    w1t = W1.T
    w2t = W2.T
    b1c = b1.reshape(HID, 1)
    b2c = b2.reshape(4, 1)

    m1 = NRAYS * NS
    m2 = NRAYS * NF
    ot = rays_o.T
    dt = rays_d.T
    bndt = bounds.T

    def col_spec(rows, cols):
        return pl.BlockSpec((rows, cols), lambda j: (0, j))

    def fix_spec(shape):
        return pl.BlockSpec(shape, lambda j: (0, 0))

    rawt1 = pl.pallas_call(
        _m1_body,
        grid=(NS // SPG1,),
        in_specs=[
            fix_spec((3, NRAYS)), fix_spec((3, NRAYS)), fix_spec((2, NRAYS)),
            pl.BlockSpec((SPG1, 1, 1), lambda j: (j, 0, 0)),
            fix_spec((HID, 3)), fix_spec((HID, 1)),
            fix_spec((4, HID)), fix_spec((4, 1)),
        ],
        out_specs=col_spec(4, SPG1 * NRAYS),
        out_shape=jax.ShapeDtypeStruct((4, m1), F32),
    )(ot, dt, bndt, t_col.reshape(NS, 1, 1), w1t, b1c, w2t, b2c)
    ch1t = [rawt1[c].reshape(NS, NRAYS) for c in range(4)]

    rgbct, zfinet = pl.pallas_call(
        _s_body,
        grid=(NRAYS // TR_S,),
        in_specs=[
            col_spec(3, TR_S), col_spec(2, TR_S),
            fix_spec((NS, 1)), fix_spec((NI, 1)),
            col_spec(NS, TR_S), col_spec(NS, TR_S), col_spec(NS, TR_S),
            col_spec(NS, TR_S),
        ],
        out_specs=[col_spec(3, TR_S), col_spec(NF, TR_S)],
        out_shape=[
            jax.ShapeDtypeStruct((3, NRAYS), F32),
            jax.ShapeDtypeStruct((NF, NRAYS), F32),
        ],
    )(dt, bndt, t_col, u_col, *ch1t)

    rawt2 = pl.pallas_call(
        _m2_body,
        grid=(NF // SPG2,),
        in_specs=[
            fix_spec((3, NRAYS)), fix_spec((3, NRAYS)),
            pl.BlockSpec((SPG2, 1, NRAYS), lambda j: (j, 0, 0)),
            fix_spec((HID, 3)), fix_spec((HID, 1)),
            fix_spec((4, HID)), fix_spec((4, 1)),
        ],
        out_specs=col_spec(4, SPG2 * NRAYS),
        out_shape=jax.ShapeDtypeStruct((4, m2), F32),
    )(ot, dt, zfinet.reshape(NF, 1, NRAYS), w1t, b1c, w2t, b2c)
    ch2t = [rawt2[c].reshape(NF, NRAYS) for c in range(4)]

    rgbt, deptht, acct, weights = pl.pallas_call(
        _r_body,
        grid=(NRAYS // TR_R,),
        in_specs=[
            col_spec(3, TR_R), col_spec(NF, TR_R),
            col_spec(NF, TR_R), col_spec(NF, TR_R), col_spec(NF, TR_R),
            col_spec(NF, TR_R),
        ],
        out_specs=[col_spec(3, TR_R), col_spec(1, TR_R), col_spec(1, TR_R),
                   pl.BlockSpec((TR_R, NF), lambda j: (j, 0))],
        out_shape=[
            jax.ShapeDtypeStruct((3, NRAYS), F32),
            jax.ShapeDtypeStruct((1, NRAYS), F32),
            jax.ShapeDtypeStruct((1, NRAYS), F32),
            jax.ShapeDtypeStruct((NRAYS, NF), F32),
        ],
    )(dt, zfinet, *ch2t)

    return (rgbct.T, rgbt.T, deptht.reshape(NRAYS), acct.reshape(NRAYS),
            weights)


# speedup vs baseline: 1.0132x; 1.0132x over previous
import jax
import jax.numpy as jnp
from jax import lax
from jax.experimental import pallas as pl

NRAYS = 4096
NS = 64
NI = 128
NF = NS + NI
HID = 256

SPG1 = 4
SPG2 = 4
TR_S = 128
TR_R = 128

F32 = jnp.float32
I32 = jnp.int32


def _sigmoid(x):
    return 1.0 / (1.0 + jnp.exp(-x))


def _mlp_t(pts_t, w1t, b1c, w2t, b2c):
    h = jnp.maximum(
        jnp.dot(w1t, pts_t, preferred_element_type=F32) + b1c, 0.0)
    return jnp.dot(w2t, h, preferred_element_type=F32) + b2c


def _m1_body(ot_ref, dt_ref, bndt_ref, t_ref, w1t_ref, b1c_ref,
             w2t_ref, b2c_ref, rawt_ref):
    ot = ot_ref[...]
    dt = dt_ref[...]
    near = bndt_ref[...][0:1, :]
    far = bndt_ref[...][1:2, :]
    tv = jnp.squeeze(t_ref[...], axis=1)
    cols = []
    for g in range(SPG1):
        tg = tv[g:g + 1, :]
        zc = near * (1.0 - tg) + far * tg
        cols.append(ot + dt * zc)
    pts_t = jnp.concatenate(cols, axis=1)
    rawt_ref[...] = _mlp_t(pts_t, w1t_ref[...], b1c_ref[...],
                           w2t_ref[...], b2c_ref[...])


def _m2_body(ot_ref, dt_ref, zct_ref, w1t_ref, b1c_ref, w2t_ref, b2c_ref,
             rawt_ref):
    ot = ot_ref[...]
    dt = dt_ref[...]
    zc = jnp.squeeze(zct_ref[...], axis=1)
    cols = [ot + dt * zc[g:g + 1, :] for g in range(SPG2)]
    pts_t = jnp.concatenate(cols, axis=1)
    rawt_ref[...] = _mlp_t(pts_t, w1t_ref[...], b1c_ref[...],
                           w2t_ref[...], b2c_ref[...])


def _cumsum_sub(x, n):
    k = 1
    while k < n:
        shifted = jnp.concatenate(
            [jnp.zeros((k,) + x.shape[1:], x.dtype), x[: n - k]], axis=0)
        x = x + shifted
        k *= 2
    return x


def _s_body(d_ref, bnd_ref, t_ref, u_ref, r0_ref, r1_ref, r2_ref, sg_ref,
            rgbc_ref, zf_ref):
    tr = TR_S
    d = d_ref[...]
    near = bnd_ref[...][0:1, :]
    far = bnd_ref[...][1:2, :]
    t = t_ref[...]
    z = near * (1.0 - t) + far * t

    dnorm = jnp.sqrt(jnp.sum(d * d, axis=0, keepdims=True))
    sigma = jnp.maximum(sg_ref[...], 0.0)
    dz = z[1:, :] - z[:-1, :]
    dists = jnp.concatenate([dz, jnp.full((1, tr), 1e10, F32)], axis=0)
    dists = dists * dnorm
    e = jnp.exp(-sigma * dists)
    alpha = 1.0 - e
    lt = jnp.log(e + 1e-10)
    ct_inc = _cumsum_sub(lt, NS)
    ct_exc = jnp.concatenate([jnp.zeros((1, tr), F32), ct_inc[:-1, :]], axis=0)
    trans = jnp.exp(ct_exc)
    weights = alpha * trans
    rgb_rows = [jnp.sum(weights * _sigmoid(rc_ref[...]), axis=0, keepdims=True)
                for rc_ref in (r0_ref, r1_ref, r2_ref)]
    rgbc_ref[...] = jnp.concatenate(rgb_rows, axis=0)

    w = weights + 1e-5
    pdf = w / jnp.sum(w, axis=0, keepdims=True)
    cdf = _cumsum_sub(pdf, NS)
    u = u_ref[...]

    m3 = (cdf[:, None, :] <= u[None, :, :]).astype(F32)
    zpad1 = jnp.zeros((1, tr), F32)
    g1c = jnp.concatenate([pdf[1:, :], zpad1], axis=0)
    b0c = jnp.concatenate([dz, zpad1], axis=0)
    b1c = jnp.concatenate([dz[1:, :], zpad1, zpad1], axis=0)
    g0 = jnp.sum(pdf[:, None, :] * m3, axis=0)
    g1 = cdf[0:1, :] + jnp.sum(g1c[:, None, :] * m3, axis=0)
    bb0 = z[0:1, :] + jnp.sum(b0c[:, None, :] * m3, axis=0)
    bb1 = z[1:2, :] + jnp.sum(b1c[:, None, :] * m3, axis=0)
    denom = g1 - g0
    denom = jnp.where(denom < 1e-5, 1.0, denom)
    tt = (u - g0) / denom
    s = bb0 + tt * (bb1 - bb0)

    big = jnp.full((NS, tr), 3e38, F32)
    c = jnp.concatenate([z, big, s], axis=0)
    row = lax.broadcasted_iota(I32, (256, 1), 0)
    for k in (128, 64, 32, 16, 8, 4, 2, 1):
        keep = (row & k) == 0
        down = jnp.concatenate([c[k:, :], c[:k, :]], axis=0)
        up = jnp.concatenate([c[256 - k:, :], c[:256 - k, :]], axis=0)
        partner = jnp.where(keep, down, up)
        mn = jnp.minimum(c, partner)
        mx = jnp.maximum(c, partner)
        c = jnp.where(keep, mn, mx)
    zf_ref[...] = c[:NF, :]


def _r_body(d_ref, zf_ref, r0_ref, r1_ref, r2_ref, sg_ref,
            rgb_ref, depth_ref, acc_ref, wout_ref):
    tr = TR_R
    d = d_ref[...]
    zf = zf_ref[...]
    dnorm = jnp.sqrt(jnp.sum(d * d, axis=0, keepdims=True))
    sigma = jnp.maximum(sg_ref[...], 0.0)
    dz = zf[1:, :] - zf[:-1, :]
    dists = jnp.concatenate([dz, jnp.full((1, tr), 1e10, F32)], axis=0)
    dists = dists * dnorm
    e = jnp.exp(-sigma * dists)
    alpha = 1.0 - e
    lt = jnp.log(e + 1e-10)
    ct_inc = _cumsum_sub(lt, NF)
    ct_exc = jnp.concatenate([jnp.zeros((1, tr), F32), ct_inc[:-1, :]], axis=0)
    trans = jnp.exp(ct_exc)
    weights = alpha * trans
    rgb_rows = [jnp.sum(weights * _sigmoid(rc_ref[...]), axis=0, keepdims=True)
                for rc_ref in (r0_ref, r1_ref, r2_ref)]
    rgb_ref[...] = jnp.concatenate(rgb_rows, axis=0)
    depth_ref[...] = jnp.sum(weights * zf, axis=0, keepdims=True)
    acc_ref[...] = jnp.sum(weights, axis=0, keepdims=True)
    wout_ref[...] = weights


@jax.jit
def kernel(rays_o, rays_d, bounds, W1, b1, W2, b2):
    t_vals = jnp.linspace(0.0, 1.0, NS, dtype=F32).reshape(1, NS)
    t_col = t_vals.reshape(NS, 1)
    u_col = jnp.linspace(0.0, 1.0, NI, dtype=F32)[::-1].reshape(NI, 1)
    w1t = W1.T
    w2t = W2.T
    b1c = b1.reshape(HID, 1)
    b2c = b2.reshape(4, 1)

    m1 = NRAYS * NS
    m2 = NRAYS * NF
    ot = rays_o.T
    dt = rays_d.T
    bndt = bounds.T

    def col_spec(rows, cols):
        return pl.BlockSpec((rows, cols), lambda j: (0, j))

    def fix_spec(shape):
        return pl.BlockSpec(shape, lambda j: (0, 0))

    rawt1 = pl.pallas_call(
        _m1_body,
        grid=(NS // SPG1,),
        in_specs=[
            fix_spec((3, NRAYS)), fix_spec((3, NRAYS)), fix_spec((2, NRAYS)),
            pl.BlockSpec((SPG1, 1, 1), lambda j: (j, 0, 0)),
            fix_spec((HID, 3)), fix_spec((HID, 1)),
            fix_spec((4, HID)), fix_spec((4, 1)),
        ],
        out_specs=col_spec(4, SPG1 * NRAYS),
        out_shape=jax.ShapeDtypeStruct((4, m1), F32),
    )(ot, dt, bndt, t_col.reshape(NS, 1, 1), w1t, b1c, w2t, b2c)
    ch1t = [rawt1[c].reshape(NS, NRAYS) for c in range(4)]

    rgbct, zfinet = pl.pallas_call(
        _s_body,
        grid=(NRAYS // TR_S,),
        in_specs=[
            col_spec(3, TR_S), col_spec(2, TR_S),
            fix_spec((NS, 1)), fix_spec((NI, 1)),
            col_spec(NS, TR_S), col_spec(NS, TR_S), col_spec(NS, TR_S),
            col_spec(NS, TR_S),
        ],
        out_specs=[col_spec(3, TR_S), col_spec(NF, TR_S)],
        out_shape=[
            jax.ShapeDtypeStruct((3, NRAYS), F32),
            jax.ShapeDtypeStruct((NF, NRAYS), F32),
        ],
    )(dt, bndt, t_col, u_col, *ch1t)

    rawt2 = pl.pallas_call(
        _m2_body,
        grid=(NF // SPG2,),
        in_specs=[
            fix_spec((3, NRAYS)), fix_spec((3, NRAYS)),
            pl.BlockSpec((SPG2, 1, NRAYS), lambda j: (j, 0, 0)),
            fix_spec((HID, 3)), fix_spec((HID, 1)),
            fix_spec((4, HID)), fix_spec((4, 1)),
        ],
        out_specs=col_spec(4, SPG2 * NRAYS),
        out_shape=jax.ShapeDtypeStruct((4, m2), F32),
    )(ot, dt, zfinet.reshape(NF, 1, NRAYS), w1t, b1c, w2t, b2c)
    ch2t = [rawt2[c].reshape(NF, NRAYS) for c in range(4)]

    rgbt, deptht, acct, weightst = pl.pallas_call(
        _r_body,
        grid=(NRAYS // TR_R,),
        in_specs=[
            col_spec(3, TR_R), col_spec(NF, TR_R),
            col_spec(NF, TR_R), col_spec(NF, TR_R), col_spec(NF, TR_R),
            col_spec(NF, TR_R),
        ],
        out_specs=[col_spec(3, TR_R), col_spec(1, TR_R), col_spec(1, TR_R),
                   col_spec(NF, TR_R)],
        out_shape=[
            jax.ShapeDtypeStruct((3, NRAYS), F32),
            jax.ShapeDtypeStruct((1, NRAYS), F32),
            jax.ShapeDtypeStruct((1, NRAYS), F32),
            jax.ShapeDtypeStruct((NF, NRAYS), F32),
        ],
    )(dt, zfinet, *ch2t)

    return (rgbct.T, rgbt.T, deptht.reshape(NRAYS), acct.reshape(NRAYS),
            weightst.T)


# speedup vs baseline: 1.0248x; 1.0115x over previous
import jax
import jax.numpy as jnp
from jax import lax
from jax.experimental import pallas as pl

NRAYS = 4096
NS = 64
NI = 128
NF = NS + NI
HID = 256

SPG1 = 8
SPG2 = 8
TR_S = 128
TR_R = 128

F32 = jnp.float32
I32 = jnp.int32


def _sigmoid(x):
    return 1.0 / (1.0 + jnp.exp(-x))


def _mlp_t(pts_t, w1t, b1c, w2t, b2c):
    h = jnp.maximum(
        jnp.dot(w1t, pts_t, preferred_element_type=F32) + b1c, 0.0)
    return jnp.dot(w2t, h, preferred_element_type=F32) + b2c


def _m1_body(ot_ref, dt_ref, bndt_ref, t_ref, w1t_ref, b1c_ref,
             w2t_ref, b2c_ref, rawt_ref):
    ot = ot_ref[...]
    dt = dt_ref[...]
    near = bndt_ref[...][0:1, :]
    far = bndt_ref[...][1:2, :]
    tv = jnp.squeeze(t_ref[...], axis=1)
    cols = []
    for g in range(SPG1):
        tg = tv[g:g + 1, :]
        zc = near * (1.0 - tg) + far * tg
        cols.append(ot + dt * zc)
    pts_t = jnp.concatenate(cols, axis=1)
    rawt_ref[...] = _mlp_t(pts_t, w1t_ref[...], b1c_ref[...],
                           w2t_ref[...], b2c_ref[...])


def _m2_body(ot_ref, dt_ref, zct_ref, w1t_ref, b1c_ref, w2t_ref, b2c_ref,
             rawt_ref):
    ot = ot_ref[...]
    dt = dt_ref[...]
    zc = jnp.squeeze(zct_ref[...], axis=1)
    cols = [ot + dt * zc[g:g + 1, :] for g in range(SPG2)]
    pts_t = jnp.concatenate(cols, axis=1)
    rawt_ref[...] = _mlp_t(pts_t, w1t_ref[...], b1c_ref[...],
                           w2t_ref[...], b2c_ref[...])


def _cumsum_sub(x, n):
    k = 1
    while k < n:
        shifted = jnp.concatenate(
            [jnp.zeros((k,) + x.shape[1:], x.dtype), x[: n - k]], axis=0)
        x = x + shifted
        k *= 2
    return x


def _s_body(d_ref, bnd_ref, t_ref, u_ref, r0_ref, r1_ref, r2_ref, sg_ref,
            rgbc_ref, zf_ref):
    tr = TR_S
    d = d_ref[...]
    near = bnd_ref[...][0:1, :]
    far = bnd_ref[...][1:2, :]
    t = t_ref[...]
    z = near * (1.0 - t) + far * t

    dnorm = jnp.sqrt(jnp.sum(d * d, axis=0, keepdims=True))
    sigma = jnp.maximum(sg_ref[...], 0.0)
    dz = z[1:, :] - z[:-1, :]
    dists = jnp.concatenate([dz, jnp.full((1, tr), 1e10, F32)], axis=0)
    dists = dists * dnorm
    e = jnp.exp(-sigma * dists)
    alpha = 1.0 - e
    lt = jnp.log(e + 1e-10)
    ct_inc = _cumsum_sub(lt, NS)
    ct_exc = jnp.concatenate([jnp.zeros((1, tr), F32), ct_inc[:-1, :]], axis=0)
    trans = jnp.exp(ct_exc)
    weights = alpha * trans
    rgb_rows = [jnp.sum(weights * _sigmoid(rc_ref[...]), axis=0, keepdims=True)
                for rc_ref in (r0_ref, r1_ref, r2_ref)]
    rgbc_ref[...] = jnp.concatenate(rgb_rows, axis=0)

    w = weights + 1e-5
    pdf = w / jnp.sum(w, axis=0, keepdims=True)
    cdf = _cumsum_sub(pdf, NS)
    u = u_ref[...]

    m3 = (cdf[:, None, :] <= u[None, :, :]).astype(F32)
    zpad1 = jnp.zeros((1, tr), F32)
    g1c = jnp.concatenate([pdf[1:, :], zpad1], axis=0)
    b0c = jnp.concatenate([dz, zpad1], axis=0)
    b1c = jnp.concatenate([dz[1:, :], zpad1, zpad1], axis=0)
    g0 = jnp.sum(pdf[:, None, :] * m3, axis=0)
    g1 = cdf[0:1, :] + jnp.sum(g1c[:, None, :] * m3, axis=0)
    bb0 = z[0:1, :] + jnp.sum(b0c[:, None, :] * m3, axis=0)
    bb1 = z[1:2, :] + jnp.sum(b1c[:, None, :] * m3, axis=0)
    denom = g1 - g0
    denom = jnp.where(denom < 1e-5, 1.0, denom)
    tt = (u - g0) / denom
    s = bb0 + tt * (bb1 - bb0)

    big = jnp.full((NS, tr), 3e38, F32)
    c = jnp.concatenate([z, big, s], axis=0)
    row = lax.broadcasted_iota(I32, (256, 1), 0)
    for k in (128, 64, 32, 16, 8, 4, 2, 1):
        keep = (row & k) == 0
        down = jnp.concatenate([c[k:, :], c[:k, :]], axis=0)
        up = jnp.concatenate([c[256 - k:, :], c[:256 - k, :]], axis=0)
        partner = jnp.where(keep, down, up)
        mn = jnp.minimum(c, partner)
        mx = jnp.maximum(c, partner)
        c = jnp.where(keep, mn, mx)
    zf_ref[...] = c[:NF, :]


def _r_body(d_ref, zf_ref, r0_ref, r1_ref, r2_ref, sg_ref,
            rgb_ref, depth_ref, acc_ref, wout_ref):
    tr = TR_R
    d = d_ref[...]
    zf = zf_ref[...]
    dnorm = jnp.sqrt(jnp.sum(d * d, axis=0, keepdims=True))
    sigma = jnp.maximum(sg_ref[...], 0.0)
    dz = zf[1:, :] - zf[:-1, :]
    dists = jnp.concatenate([dz, jnp.full((1, tr), 1e10, F32)], axis=0)
    dists = dists * dnorm
    e = jnp.exp(-sigma * dists)
    alpha = 1.0 - e
    lt = jnp.log(e + 1e-10)
    ct_inc = _cumsum_sub(lt, NF)
    ct_exc = jnp.concatenate([jnp.zeros((1, tr), F32), ct_inc[:-1, :]], axis=0)
    trans = jnp.exp(ct_exc)
    weights = alpha * trans
    rgb_rows = [jnp.sum(weights * _sigmoid(rc_ref[...]), axis=0, keepdims=True)
                for rc_ref in (r0_ref, r1_ref, r2_ref)]
    rgb_ref[...] = jnp.concatenate(rgb_rows, axis=0)
    depth_ref[...] = jnp.sum(weights * zf, axis=0, keepdims=True)
    acc_ref[...] = jnp.sum(weights, axis=0, keepdims=True)
    wout_ref[...] = weights


@jax.jit
def kernel(rays_o, rays_d, bounds, W1, b1, W2, b2):
    t_vals = jnp.linspace(0.0, 1.0, NS, dtype=F32).reshape(1, NS)
    t_col = t_vals.reshape(NS, 1)
    u_col = jnp.linspace(0.0, 1.0, NI, dtype=F32)[::-1].reshape(NI, 1)
    w1t = W1.T
    w2t = W2.T
    b1c = b1.reshape(HID, 1)
    b2c = b2.reshape(4, 1)

    m1 = NRAYS * NS
    m2 = NRAYS * NF
    ot = rays_o.T
    dt = rays_d.T
    bndt = bounds.T

    def col_spec(rows, cols):
        return pl.BlockSpec((rows, cols), lambda j: (0, j))

    def fix_spec(shape):
        return pl.BlockSpec(shape, lambda j: (0, 0))

    rawt1 = pl.pallas_call(
        _m1_body,
        grid=(NS // SPG1,),
        in_specs=[
            fix_spec((3, NRAYS)), fix_spec((3, NRAYS)), fix_spec((2, NRAYS)),
            pl.BlockSpec((SPG1, 1, 1), lambda j: (j, 0, 0)),
            fix_spec((HID, 3)), fix_spec((HID, 1)),
            fix_spec((4, HID)), fix_spec((4, 1)),
        ],
        out_specs=col_spec(4, SPG1 * NRAYS),
        out_shape=jax.ShapeDtypeStruct((4, m1), F32),
    )(ot, dt, bndt, t_col.reshape(NS, 1, 1), w1t, b1c, w2t, b2c)
    ch1t = [rawt1[c].reshape(NS, NRAYS) for c in range(4)]

    rgbct, zfinet = pl.pallas_call(
        _s_body,
        grid=(NRAYS // TR_S,),
        in_specs=[
            col_spec(3, TR_S), col_spec(2, TR_S),
            fix_spec((NS, 1)), fix_spec((NI, 1)),
            col_spec(NS, TR_S), col_spec(NS, TR_S), col_spec(NS, TR_S),
            col_spec(NS, TR_S),
        ],
        out_specs=[col_spec(3, TR_S), col_spec(NF, TR_S)],
        out_shape=[
            jax.ShapeDtypeStruct((3, NRAYS), F32),
            jax.ShapeDtypeStruct((NF, NRAYS), F32),
        ],
    )(dt, bndt, t_col, u_col, *ch1t)

    rawt2 = pl.pallas_call(
        _m2_body,
        grid=(NF // SPG2,),
        in_specs=[
            fix_spec((3, NRAYS)), fix_spec((3, NRAYS)),
            pl.BlockSpec((SPG2, 1, NRAYS), lambda j: (j, 0, 0)),
            fix_spec((HID, 3)), fix_spec((HID, 1)),
            fix_spec((4, HID)), fix_spec((4, 1)),
        ],
        out_specs=col_spec(4, SPG2 * NRAYS),
        out_shape=jax.ShapeDtypeStruct((4, m2), F32),
    )(ot, dt, zfinet.reshape(NF, 1, NRAYS), w1t, b1c, w2t, b2c)
    ch2t = [rawt2[c].reshape(NF, NRAYS) for c in range(4)]

    rgbt, deptht, acct, weightst = pl.pallas_call(
        _r_body,
        grid=(NRAYS // TR_R,),
        in_specs=[
            col_spec(3, TR_R), col_spec(NF, TR_R),
            col_spec(NF, TR_R), col_spec(NF, TR_R), col_spec(NF, TR_R),
            col_spec(NF, TR_R),
        ],
        out_specs=[col_spec(3, TR_R), col_spec(1, TR_R), col_spec(1, TR_R),
                   col_spec(NF, TR_R)],
        out_shape=[
            jax.ShapeDtypeStruct((3, NRAYS), F32),
            jax.ShapeDtypeStruct((1, NRAYS), F32),
            jax.ShapeDtypeStruct((1, NRAYS), F32),
            jax.ShapeDtypeStruct((NF, NRAYS), F32),
        ],
    )(dt, zfinet, *ch2t)

    return (rgbct.T, rgbt.T, deptht.reshape(NRAYS), acct.reshape(NRAYS),
            weightst.T)
